# baseline (speedup 1.0000x reference)
import functools

import jax
import jax.numpy as jnp
from jax import lax
from jax.experimental import pallas as pl
from jax.experimental.pallas import tpu as pltpu
from jax.experimental.pallas import tpu_sc as plsc

ROWS = 64
N = 32768
L = 16
NCHUNK = N // L
GCH = 16
GELT = GCH * L
GSH = 8
NGRP = NCHUNK // GCH
NW = 32
ROWS_PER_W = ROWS // NW
N_GRID = 4
N_MICHELOT = 2
NMID = 7

_mesh = plsc.VectorSubcoreMesh(core_axis_name="c", subcore_axis_name="s")


@functools.partial(
    pl.kernel,
    out_type=jax.ShapeDtypeStruct((ROWS, N), jnp.float32),
    mesh=_mesh,
    scratch_types=[
        pltpu.VMEM((ROWS_PER_W, N + GELT), jnp.float32),
        pltpu.VMEM((16 * 2 * L,), jnp.float32),
        pltpu.SMEM((NGRP + 1,), jnp.float32),
        pltpu.SMEM((NGRP + 8,), jnp.int32),
        pltpu.SemaphoreType.DMA,
        pltpu.SemaphoreType.DMA,
    ],
)
def _sparsemax_sc(x_hbm, out_hbm, xbuf, rot, gmax_s, gb_s, sem0, sem1):
    wid = lax.axis_index("s") * 2 + lax.axis_index("c")
    row0 = wid * ROWS_PER_W

    in0 = pltpu.async_copy(x_hbm.at[row0], xbuf.at[0, pl.ds(0, N)], sem0)
    in1 = pltpu.async_copy(x_hbm.at[row0 + 1], xbuf.at[1, pl.ds(0, N)], sem1)

    def allreduce_multi(vs, comb):
        vs = list(vs)
        for sh in (8, 4, 2, 1):
            for q, v in enumerate(vs):
                rot[pl.ds(q * 2 * L, L)] = v
                rot[pl.ds(q * 2 * L + L, L)] = v
            for q, v in enumerate(vs):
                vs[q] = comb(v, rot[pl.ds(q * 2 * L + sh, L)])
        return vs

    def process_row(r):
        neg = jnp.full((L,), -3.4e38, dtype=jnp.float32)
        for j in range(GCH):
            xbuf[r, pl.ds(N + j * L, L)] = neg
        gmax_s[NGRP] = neg[0]

        def grp_body(i, gacc):
            gvs = []
            for q in range(4):
                base = (i * 4 + q) * GELT
                gv = xbuf[r, pl.ds(base, L)]
                for j in range(1, GCH):
                    gv = jnp.maximum(gv, xbuf[r, pl.ds(base + j * L, L)])
                gvs.append(gv)
            gvs = allreduce_multi(gvs, jnp.maximum)
            for q in range(4):
                gmax_s[i * 4 + q] = gvs[q][0]
            return jnp.maximum(jnp.maximum(gvs[0], gvs[1]),
                               jnp.maximum(gvs[2], jnp.maximum(gvs[3], gacc)))

        gacc = lax.fori_loop(0, NGRP // 4, grp_body,
                             jnp.full((L,), -3.4e38, dtype=jnp.float32))
        lo = gacc - 1.0

        def list_body(g, offg):
            pv = jnp.where(gmax_s[g] >= lo, 1.0, 0.0)
            gb_s[offg] = g
            return offg + lax.convert_element_type(pv[0], jnp.int32)

        ncg = lax.fori_loop(0, NGRP, list_body, jnp.int32(0))

        def pad_list(n):
            for q in range(8):
                gb_s[n + q] = NGRP
        pad_list(ncg)

        def reprune(blo, n):
            def rb(t, off):
                for u in range(4):
                    g = gb_s[t * 4 + u]
                    pv = jnp.where(gmax_s[g] >= blo, 1.0, 0.0)
                    gb_s[off] = g
                    off = off + lax.convert_element_type(pv[0], jnp.int32)
                return off

            n2 = lax.fori_loop(0, lax.shift_right_logical(n + 3, 2),
                               rb, jnp.int32(0))
            pad_list(n2)
            return n2

        def michelot_step(tau, n):
            def mbody(t, carry):
                s, k = carry
                for u in range(8):
                    base = gb_s[t * 8 + u] * GELT
                    for j in range(GCH):
                        c = xbuf[r, pl.ds(base + j * L, L)]
                        sel = c > tau
                        s = s + jnp.where(sel, c, 0.0)
                        k = k + jnp.where(sel, 1.0, 0.0)
                return s, k

            z = jnp.zeros((L,), jnp.float32)
            mblk = lax.shift_right_logical(n + 7, 3)
            s, k = lax.fori_loop(0, mblk, mbody, (z, z))
            s, k = allreduce_multi([s, k], jnp.add)
            return (s - 1.0) / k

        blo0 = michelot_step(lo, ncg)
        ncg = reprune(blo0, ncg)
        w_pre = lo + 1.0 - blo0

        def grid_pass(_, carry):
            blo, w, n = carry
            step = w * 0.125
            ksteps = [step * float(k) for k in range(1, NMID + 1)]

            def scan(t, accs):
                out = list(accs)
                for u in range(4):
                    base = gb_s[t * 4 + u] * GELT
                    for j in range(GCH):
                        d = xbuf[r, pl.ds(base + j * L, L)] - blo
                        for k in range(NMID):
                            out[k] = out[k] + jnp.maximum(d - ksteps[k], 0.0)
                return tuple(out)

            z = jnp.zeros((L,), jnp.float32)
            nblk = lax.shift_right_logical(n + 3, 2)
            accs = lax.fori_loop(0, nblk, scan, (z,) * NMID)
            sums = allreduce_multi(accs, jnp.add)
            cnt = jnp.zeros((L,), jnp.float32)
            for k in range(NMID):
                cnt = cnt + jnp.where(sums[k] >= 1.0, 1.0, 0.0)
            blo = blo + cnt * step
            return blo, step, reprune(blo, n)

        blo, _, ncg = lax.fori_loop(0, N_GRID, grid_pass, (blo0, w_pre, ncg))

        def michelot(_, tau):
            def mbody(t, carry):
                s, k = carry
                for u in range(8):
                    base = gb_s[t * 8 + u] * GELT
                    for j in range(GCH):
                        c = xbuf[r, pl.ds(base + j * L, L)]
                        sel = c > tau
                        s = s + jnp.where(sel, c, 0.0)
                        k = k + jnp.where(sel, 1.0, 0.0)
                return s, k

            z = jnp.zeros((L,), jnp.float32)
            mblk = lax.shift_right_logical(ncg + 7, 3)
            s, k = lax.fori_loop(0, mblk, mbody, (z, z))
            s, k = allreduce_multi([s, k], jnp.add)
            return (s - 1.0) / k

        tau = lax.fori_loop(0, N_MICHELOT, michelot, blo)

        def out_body(g, _):
            base = g * GELT
            for j in range(GCH):
                v = xbuf[r, pl.ds(base + j * L, L)]
                xbuf[r, pl.ds(base + j * L, L)] = jnp.maximum(v - tau, 0.0)
            return 0

        lax.fori_loop(0, NGRP, out_body, 0)

    in0.wait()
    process_row(0)
    o0 = pltpu.async_copy(xbuf.at[0, pl.ds(0, N)], out_hbm.at[row0], sem0)
    in1.wait()
    process_row(1)
    o1 = pltpu.async_copy(xbuf.at[1, pl.ds(0, N)], out_hbm.at[row0 + 1], sem1)
    o0.wait()
    o1.wait()


def kernel(x):
    return _sparsemax_sc(x)


# speedup vs baseline: 1.0272x; 1.0272x over previous
import functools

import jax
import jax.numpy as jnp
from jax import lax
from jax.experimental import pallas as pl
from jax.experimental.pallas import tpu as pltpu
from jax.experimental.pallas import tpu_sc as plsc

ROWS = 64
N = 32768
L = 16
NCHUNK = N // L
GCH = 16
GELT = GCH * L
GSH = 8
NGRP = NCHUNK // GCH
NW = 32
ROWS_PER_W = ROWS // NW
N_GRID = 4
N_MICHELOT = 2
NMID = 7

_mesh = plsc.VectorSubcoreMesh(core_axis_name="c", subcore_axis_name="s")


@functools.partial(
    pl.kernel,
    out_type=jax.ShapeDtypeStruct((ROWS, N), jnp.float32),
    mesh=_mesh,
    scratch_types=[
        pltpu.VMEM((ROWS_PER_W, N + GELT), jnp.float32),
        pltpu.VMEM((16 * 2 * L,), jnp.float32),
        pltpu.SMEM((NGRP + 1,), jnp.float32),
        pltpu.SMEM((NGRP + 8,), jnp.int32),
        pltpu.SemaphoreType.DMA,
        pltpu.SemaphoreType.DMA,
    ],
)
def _sparsemax_sc(x_hbm, out_hbm, xbuf, rot, gmax_s, gb_s, sem0, sem1):
    wid = lax.axis_index("s") * 2 + lax.axis_index("c")
    row0 = wid * ROWS_PER_W

    in0 = pltpu.async_copy(x_hbm.at[row0], xbuf.at[0, pl.ds(0, N)], sem0)
    in1 = pltpu.async_copy(x_hbm.at[row0 + 1], xbuf.at[1, pl.ds(0, N)], sem1)

    def allreduce_multi(vs, comb):
        vs = list(vs)
        for sh in (8, 4, 2, 1):
            for q, v in enumerate(vs):
                rot[pl.ds(q * 2 * L, L)] = v
                rot[pl.ds(q * 2 * L + L, L)] = v
            for q, v in enumerate(vs):
                vs[q] = comb(v, rot[pl.ds(q * 2 * L + sh, L)])
        return vs

    def process_row(r):
        neg = jnp.full((L,), -3.4e38, dtype=jnp.float32)
        for j in range(GCH):
            xbuf[r, pl.ds(N + j * L, L)] = neg
        gmax_s[NGRP] = neg[0]

        def grp_body(i, gacc):
            gvs = []
            for q in range(4):
                base = (i * 4 + q) * GELT
                gv = xbuf[r, pl.ds(base, L)]
                for j in range(1, GCH):
                    gv = jnp.maximum(gv, xbuf[r, pl.ds(base + j * L, L)])
                gvs.append(gv)
            gvs = allreduce_multi(gvs, jnp.maximum)
            for q in range(4):
                gmax_s[i * 4 + q] = gvs[q][0]
            return jnp.maximum(jnp.maximum(gvs[0], gvs[1]),
                               jnp.maximum(gvs[2], jnp.maximum(gvs[3], gacc)))

        gacc = lax.fori_loop(0, NGRP // 4, grp_body,
                             jnp.full((L,), -3.4e38, dtype=jnp.float32))
        lo = gacc - 1.0

        def list_body(g, offg):
            pv = jnp.where(gmax_s[g] >= lo, 1.0, 0.0)
            gb_s[offg] = g
            return offg + lax.convert_element_type(pv[0], jnp.int32)

        ncg = lax.fori_loop(0, NGRP, list_body, jnp.int32(0))

        def pad_list(n):
            for q in range(8):
                gb_s[n + q] = NGRP
        pad_list(ncg)

        def reprune(blo, n):
            def rb(t, off):
                for u in range(4):
                    g = gb_s[t * 4 + u]
                    pv = jnp.where(gmax_s[g] >= blo, 1.0, 0.0)
                    gb_s[off] = g
                    off = off + lax.convert_element_type(pv[0], jnp.int32)
                return off

            n2 = lax.fori_loop(0, lax.shift_right_logical(n + 3, 2),
                               rb, jnp.int32(0))
            pad_list(n2)
            return n2

        def grid_pass(_, carry):
            blo, w, n = carry
            step = w * 0.125
            ksteps = [step * float(k) for k in range(1, NMID + 1)]

            def scan(t, accs):
                out = list(accs)
                for u in range(4):
                    base = gb_s[t * 4 + u] * GELT
                    for j in range(GCH):
                        d = xbuf[r, pl.ds(base + j * L, L)] - blo
                        for k in range(NMID):
                            out[k] = out[k] + jnp.maximum(d - ksteps[k], 0.0)
                return tuple(out)

            z = jnp.zeros((L,), jnp.float32)
            nblk = lax.shift_right_logical(n + 3, 2)
            accs = lax.fori_loop(0, nblk, scan, (z,) * NMID)
            sums = allreduce_multi(accs, jnp.add)
            cnt = jnp.zeros((L,), jnp.float32)
            for k in range(NMID):
                cnt = cnt + jnp.where(sums[k] >= 1.0, 1.0, 0.0)
            blo = blo + cnt * step
            return blo, step, reprune(blo, n)

        w0 = jnp.full((L,), 1.0, dtype=jnp.float32)
        blo, _, ncg = lax.fori_loop(0, N_GRID, grid_pass, (lo, w0, ncg))

        def michelot(_, tau):
            def mbody(t, carry):
                s, k = carry
                for u in range(8):
                    base = gb_s[t * 8 + u] * GELT
                    for j in range(GCH):
                        c = xbuf[r, pl.ds(base + j * L, L)]
                        sel = c > tau
                        s = s + jnp.where(sel, c, 0.0)
                        k = k + jnp.where(sel, 1.0, 0.0)
                return s, k

            z = jnp.zeros((L,), jnp.float32)
            mblk = lax.shift_right_logical(ncg + 7, 3)
            s, k = lax.fori_loop(0, mblk, mbody, (z, z))
            s, k = allreduce_multi([s, k], jnp.add)
            return (s - 1.0) / k

        tau = lax.fori_loop(0, N_MICHELOT, michelot, blo)

        def out_body(g, _):
            base = g * GELT
            for j in range(GCH):
                v = xbuf[r, pl.ds(base + j * L, L)]
                xbuf[r, pl.ds(base + j * L, L)] = jnp.maximum(v - tau, 0.0)
            return 0

        lax.fori_loop(0, NGRP, out_body, 0)

    in0.wait()
    process_row(0)
    o0 = pltpu.async_copy(xbuf.at[0, pl.ds(0, N)], out_hbm.at[row0], sem0)
    in1.wait()
    process_row(1)
    o1 = pltpu.async_copy(xbuf.at[1, pl.ds(0, N)], out_hbm.at[row0 + 1], sem1)
    o0.wait()
    o1.wait()


def kernel(x):
    return _sparsemax_sc(x)
